# baseline (speedup 1.0000x reference)
import functools

import numpy as np
import jax
from jax import lax
import jax.numpy as jnp
from jax.experimental import pallas as pl
from jax.experimental.pallas import tpu as pltpu
from jax.experimental.pallas import tpu_sc as plsc

_B, _T, _N = 16, 16, 1024
_F = _B * _T
_NUM_STRATA = 4
_GRADIENT_STRENGTH = 0.15
_REGION_RATIOS = [0.4, 0.3, 0.2, 0.1]
_MASK_RATIO_STATIC = 0.75
_L = 16
_C = _N // _L
_NW = 32
_FPW = _F // _NW


def _rotl(x, d):
    return ((x << np.uint32(d)) | (x >> np.uint32(32 - d))).astype(np.uint32)


_TF_ROTS = ((13, 15, 26, 6), (17, 29, 16, 24))


def _threefry2x32(k0, k1, x0, x1):
    x0 = np.atleast_1d(x0).astype(np.uint32)
    x1 = np.atleast_1d(x1).astype(np.uint32)
    ks = (np.uint32(k0), np.uint32(k1),
          np.uint32(np.uint32(k0) ^ np.uint32(k1) ^ np.uint32(0x1BD11BDA)))
    x0 += ks[0]
    x1 += ks[1]
    inj = ((ks[1], ks[2]), (ks[2], ks[0]), (ks[0], ks[1]),
           (ks[1], ks[2]), (ks[2], ks[0]))
    for r in range(5):
        for rot in _TF_ROTS[r % 2]:
            x0 = (x0 + x1).astype(np.uint32)
            x1 = _rotl(x1, rot) ^ x0
        a, b = inj[r]
        x0 = (x0 + a).astype(np.uint32)
        x1 = (x1 + b + np.uint32(r + 1)).astype(np.uint32)
    return x0, x1


def _tf_fold_in(keypair, data):
    x0, x1 = _threefry2x32(keypair[0], keypair[1],
                           np.zeros(1, np.uint32), np.full(1, data, np.uint32))
    return (x0[0], x1[0])


def _tf_uniform(keypair, shape):
    size = int(np.prod(shape))
    i64 = np.arange(size, dtype=np.uint64)
    hi = (i64 >> np.uint64(32)).astype(np.uint32)
    lo = (i64 & np.uint64(0xFFFFFFFF)).astype(np.uint32)
    b0, b1 = _threefry2x32(keypair[0], keypair[1], hi, lo)
    bits = b0 ^ b1
    floats = (bits >> np.uint32(9)) | np.uint32(0x3F800000)
    return (floats.view(np.float32) - np.float32(1.0)).reshape(shape)


def _layer_ratio_list(mask_ratio):
    step = _GRADIENT_STRENGTH
    base = mask_ratio - (_NUM_STRATA - 1) * step / 2
    ratios = []
    for i in range(_NUM_STRATA):
        r = base + (_NUM_STRATA - 1 - i) * step
        r = max(0.0, min(0.9, r))
        ratios.append(r)
    weighted = sum(r * w for r, w in zip(ratios, _REGION_RATIOS))
    if weighted > 0:
        scale = mask_ratio / weighted
        ratios = [r * scale for r in ratios]
    return ratios


@functools.lru_cache(maxsize=1)
def _rank_table_f32():
    ratios = _layer_ratio_list(_MASK_RATIO_STATIC)
    sizes = [max(1, int(_N * r)) for r in _REGION_RATIOS]
    diff = _N - sum(sizes)
    if diff != 0:
        mi = sizes.index(max(sizes))
        sizes[mi] += diff
    key = (np.uint32(0), np.uint32(42))
    tbl = np.zeros((_B, _T, _N), np.float32)
    bI = np.arange(_B)[:, None, None]
    tI = np.arange(_T)[None, :, None]
    start = 0
    for j, layer_idx in enumerate(range(_NUM_STRATA - 1, -1, -1)):
        size = sizes[layer_idx]
        start_j = start
        start += size
        if size == 0:
            continue
        num = min(int(size * ratios[layer_idx]), size)
        if num <= 0:
            continue
        u = _tf_uniform(_tf_fold_in(key, j), (_B, _T, size))
        perm = np.argsort(u, axis=-1, kind="stable")[:, :, :num]
        tbl[bI, tI, start_j + perm] = 1.0
    return tbl.reshape(_F, _N)


_TBL = _rank_table_f32()


def _sc_body(scores_hbm, tbl_hbm, out_hbm,
             s_v, t_v, o_v, key_a, key_b, idx_a, idx_b, hist):
    iota = lax.iota(jnp.int32, _L)
    ones = jnp.ones((_L,), jnp.int32)
    wid = lax.axis_index("s") * 2 + lax.axis_index("c")

    def frame_body(i, _):
        f = wid * _FPW + i
        pltpu.sync_copy(scores_hbm.at[f], s_v)
        pltpu.sync_copy(tbl_hbm.at[f], t_v)

        def key0_at(c):
            s = plsc.load_gather(s_v, [iota * _C + c])
            b = lax.bitcast_convert_type(s, jnp.int32)
            return ~jnp.where(b >= 0, b ^ jnp.int32(-(2 ** 31)), ~b)

        def zero_hist():
            for j in range(16):
                hist[pl.ds(j * _L, _L)] = jnp.zeros((_L,), jnp.int32)

        def scan_hist():
            carry = jnp.int32(0)
            for j in range(16):
                h = hist[pl.ds(j * _L, _L)]
                inc = jnp.cumsum(h)
                hist[pl.ds(j * _L, _L)] = inc - h + carry
                carry = carry + jnp.sum(h)

        zero_hist()

        def hist0_body(c2, _):
            for c in (c2 * 2, c2 * 2 + 1):
                k = key0_at(c)
                plsc.addupdate_scatter(hist, [(k & 15) * _L + iota], ones)
            return 0

        lax.fori_loop(0, _C // 2, hist0_body, 0)
        scan_hist()

        def reorder0_body(c2, _):
            for c in (c2 * 2, c2 * 2 + 1):
                k = key0_at(c)
                hidx = (k & 15) * _L + iota
                q = plsc.load_gather(hist, [hidx])
                plsc.addupdate_scatter(hist, [hidx], ones)
                dst = (q & 63) * _L + (q >> 6)
                plsc.store_scatter(key_b, [dst], k)
                plsc.store_scatter(idx_b, [dst], iota * _C + c)
            return 0

        lax.fori_loop(0, _C // 2, reorder0_body, 0)

        bufs = [(key_a, idx_a), (key_b, idx_b)]
        for p in range(1, 7):
            sh = 4 * p
            in_key, in_idx = bufs[p % 2]
            out_key, out_idx = bufs[(p + 1) % 2]
            zero_hist()

            def hist_body(c2, _, in_key=in_key, sh=sh):
                for c in (c2 * 2, c2 * 2 + 1):
                    k = in_key[pl.ds(c * _L, _L)]
                    d = (k >> sh) & 15
                    plsc.addupdate_scatter(hist, [d * _L + iota], ones)
                return 0

            lax.fori_loop(0, _C // 2, hist_body, 0)
            scan_hist()

            def reorder_body(c2, _, in_key=in_key, in_idx=in_idx,
                             out_key=out_key, out_idx=out_idx, sh=sh):
                for c in (c2 * 2, c2 * 2 + 1):
                    k = in_key[pl.ds(c * _L, _L)]
                    pidx = in_idx[pl.ds(c * _L, _L)]
                    d = (k >> sh) & 15
                    hidx = d * _L + iota
                    q = plsc.load_gather(hist, [hidx])
                    plsc.addupdate_scatter(hist, [hidx], ones)
                    dst = (q & 63) * _L + (q >> 6)
                    plsc.store_scatter(out_key, [dst], k)
                    plsc.store_scatter(out_idx, [dst], pidx)
                return 0

            lax.fori_loop(0, _C // 2, reorder_body, 0)

        zero_hist()

        def hist7_body(c2, _):
            for c in (c2 * 2, c2 * 2 + 1):
                k = key_b[pl.ds(c * _L, _L)]
                d = (k >> 28) & 15
                plsc.addupdate_scatter(hist, [d * _L + iota], ones)
            return 0

        lax.fori_loop(0, _C // 2, hist7_body, 0)
        scan_hist()

        def apply7_body(c2, _):
            for c in (c2 * 2, c2 * 2 + 1):
                k = key_b[pl.ds(c * _L, _L)]
                pidx = idx_b[pl.ds(c * _L, _L)]
                d = (k >> 28) & 15
                hidx = d * _L + iota
                q = plsc.load_gather(hist, [hidx])
                plsc.addupdate_scatter(hist, [hidx], ones)
                tv = plsc.load_gather(t_v, [q])
                plsc.store_scatter(o_v, [pidx], tv)
            return 0

        lax.fori_loop(0, _C // 2, apply7_body, 0)
        pltpu.sync_copy(o_v, out_hbm.at[f])
        return 0

    lax.fori_loop(0, _FPW, frame_body, 0)


@jax.jit
def _run(scores_flat, tbl):
    mesh = plsc.VectorSubcoreMesh(core_axis_name="c", subcore_axis_name="s")
    k = pl.kernel(
        _sc_body,
        mesh=mesh,
        compiler_params=pltpu.CompilerParams(
            needs_layout_passes=False,
            use_tc_tiling_on_sc=False,
        ),
        out_type=jax.ShapeDtypeStruct((_F, _N), jnp.float32),
        scratch_types=[
            pltpu.VMEM((_N,), jnp.float32),
            pltpu.VMEM((_N,), jnp.float32),
            pltpu.VMEM((_N,), jnp.float32),
            pltpu.VMEM((_N,), jnp.int32),
            pltpu.VMEM((_N,), jnp.int32),
            pltpu.VMEM((_N,), jnp.int32),
            pltpu.VMEM((_N,), jnp.int32),
            pltpu.VMEM((16 * _L,), jnp.int32),
        ],
    )
    return k(scores_flat, tbl)


def kernel(scores, mask_ratio):
    del mask_ratio
    scores_flat = scores.reshape(_F, _N)
    out = _run(scores_flat, jnp.asarray(_TBL))
    return (out > 0.5).reshape(_B, _T, 32, 32)


# speedup vs baseline: 1.0036x; 1.0036x over previous
import functools

import numpy as np
import jax
from jax import lax
import jax.numpy as jnp
from jax.experimental import pallas as pl
from jax.experimental.pallas import tpu as pltpu
from jax.experimental.pallas import tpu_sc as plsc

_B, _T, _N = 16, 16, 1024
_F = _B * _T
_NUM_STRATA = 4
_GRADIENT_STRENGTH = 0.15
_REGION_RATIOS = [0.4, 0.3, 0.2, 0.1]
_MASK_RATIO_STATIC = 0.75
_L = 16
_C = _N // _L
_NW = 32
_FPW = _F // _NW


def _rotl(x, d):
    return ((x << np.uint32(d)) | (x >> np.uint32(32 - d))).astype(np.uint32)


_TF_ROTS = ((13, 15, 26, 6), (17, 29, 16, 24))


def _threefry2x32(k0, k1, x0, x1):
    x0 = np.atleast_1d(x0).astype(np.uint32)
    x1 = np.atleast_1d(x1).astype(np.uint32)
    ks = (np.uint32(k0), np.uint32(k1),
          np.uint32(np.uint32(k0) ^ np.uint32(k1) ^ np.uint32(0x1BD11BDA)))
    x0 += ks[0]
    x1 += ks[1]
    inj = ((ks[1], ks[2]), (ks[2], ks[0]), (ks[0], ks[1]),
           (ks[1], ks[2]), (ks[2], ks[0]))
    for r in range(5):
        for rot in _TF_ROTS[r % 2]:
            x0 = (x0 + x1).astype(np.uint32)
            x1 = _rotl(x1, rot) ^ x0
        a, b = inj[r]
        x0 = (x0 + a).astype(np.uint32)
        x1 = (x1 + b + np.uint32(r + 1)).astype(np.uint32)
    return x0, x1


def _tf_fold_in(keypair, data):
    x0, x1 = _threefry2x32(keypair[0], keypair[1],
                           np.zeros(1, np.uint32), np.full(1, data, np.uint32))
    return (x0[0], x1[0])


def _tf_uniform(keypair, shape):
    size = int(np.prod(shape))
    i64 = np.arange(size, dtype=np.uint64)
    hi = (i64 >> np.uint64(32)).astype(np.uint32)
    lo = (i64 & np.uint64(0xFFFFFFFF)).astype(np.uint32)
    b0, b1 = _threefry2x32(keypair[0], keypair[1], hi, lo)
    bits = b0 ^ b1
    floats = (bits >> np.uint32(9)) | np.uint32(0x3F800000)
    return (floats.view(np.float32) - np.float32(1.0)).reshape(shape)


def _layer_ratio_list(mask_ratio):
    step = _GRADIENT_STRENGTH
    base = mask_ratio - (_NUM_STRATA - 1) * step / 2
    ratios = []
    for i in range(_NUM_STRATA):
        r = base + (_NUM_STRATA - 1 - i) * step
        r = max(0.0, min(0.9, r))
        ratios.append(r)
    weighted = sum(r * w for r, w in zip(ratios, _REGION_RATIOS))
    if weighted > 0:
        scale = mask_ratio / weighted
        ratios = [r * scale for r in ratios]
    return ratios


@functools.lru_cache(maxsize=1)
def _rank_table_f32():
    ratios = _layer_ratio_list(_MASK_RATIO_STATIC)
    sizes = [max(1, int(_N * r)) for r in _REGION_RATIOS]
    diff = _N - sum(sizes)
    if diff != 0:
        mi = sizes.index(max(sizes))
        sizes[mi] += diff
    key = (np.uint32(0), np.uint32(42))
    tbl = np.zeros((_B, _T, _N), np.float32)
    bI = np.arange(_B)[:, None, None]
    tI = np.arange(_T)[None, :, None]
    start = 0
    for j, layer_idx in enumerate(range(_NUM_STRATA - 1, -1, -1)):
        size = sizes[layer_idx]
        start_j = start
        start += size
        if size == 0:
            continue
        num = min(int(size * ratios[layer_idx]), size)
        if num <= 0:
            continue
        u = _tf_uniform(_tf_fold_in(key, j), (_B, _T, size))
        perm = np.argsort(u, axis=-1, kind="stable")[:, :, :num]
        tbl[bI, tI, start_j + perm] = 1.0
    return tbl.reshape(_F, _N)


_TBL = _rank_table_f32()


def _sc_body(scores_hbm, tbl_hbm, out_hbm,
             s_v0, t_v0, o_v0, key_a0, key_b0, idx_a0, idx_b0, hist0,
             s_v1, t_v1, o_v1, key_a1, key_b1, idx_a1, idx_b1, hist1):
    iota = lax.iota(jnp.int32, _L)
    ones = jnp.ones((_L,), jnp.int32)
    wid = lax.axis_index("s") * 2 + lax.axis_index("c")
    fr0 = dict(s_v=s_v0, t_v=t_v0, o_v=o_v0, key_a=key_a0, key_b=key_b0,
               idx_a=idx_a0, idx_b=idx_b0, hist=hist0)
    fr1 = dict(s_v=s_v1, t_v=t_v1, o_v=o_v1, key_a=key_a1, key_b=key_b1,
               idx_a=idx_a1, idx_b=idx_b1, hist=hist1)
    frames = (fr0, fr1)

    def key0_at(fr, c):
        s = plsc.load_gather(fr["s_v"], [iota * _C + c])
        b = lax.bitcast_convert_type(s, jnp.int32)
        return ~jnp.where(b >= 0, b ^ jnp.int32(-(2 ** 31)), ~b)

    def pair_body(i, _):
        f0 = wid * _FPW + i * 2
        for u, fr in enumerate(frames):
            pltpu.sync_copy(scores_hbm.at[f0 + u], fr["s_v"])
            pltpu.sync_copy(tbl_hbm.at[f0 + u], fr["t_v"])

        def zero_hists():
            for j in range(16):
                for fr in frames:
                    fr["hist"][pl.ds(j * _L, _L)] = jnp.zeros((_L,),
                                                              jnp.int32)

        def scan_hists():
            carries = [jnp.int32(0), jnp.int32(0)]
            for j in range(16):
                for u, fr in enumerate(frames):
                    h = fr["hist"][pl.ds(j * _L, _L)]
                    inc = jnp.cumsum(h)
                    fr["hist"][pl.ds(j * _L, _L)] = inc - h + carries[u]
                    carries[u] = carries[u] + jnp.sum(h)

        zero_hists()

        def hist0_body(c, _):
            for fr in frames:
                k = key0_at(fr, c)
                plsc.addupdate_scatter(fr["hist"], [(k & 15) * _L + iota],
                                       ones)
            return 0

        lax.fori_loop(0, _C, hist0_body, 0)
        scan_hists()

        def reorder0_body(c, _):
            for fr in frames:
                k = key0_at(fr, c)
                hidx = (k & 15) * _L + iota
                q = plsc.load_gather(fr["hist"], [hidx])
                plsc.addupdate_scatter(fr["hist"], [hidx], ones)
                dst = (q & 63) * _L + (q >> 6)
                plsc.store_scatter(fr["key_b"], [dst], k)
                plsc.store_scatter(fr["idx_b"], [dst], iota * _C + c)
            return 0

        lax.fori_loop(0, _C, reorder0_body, 0)

        for p in range(1, 7):
            sh = 4 * p
            src = "key_a" if p % 2 == 0 else "key_b"
            srcI = "idx_a" if p % 2 == 0 else "idx_b"
            dstK = "key_b" if p % 2 == 0 else "key_a"
            dstI = "idx_b" if p % 2 == 0 else "idx_a"
            zero_hists()

            def hist_body(c, _, src=src, sh=sh):
                for fr in frames:
                    k = fr[src][pl.ds(c * _L, _L)]
                    d = (k >> sh) & 15
                    plsc.addupdate_scatter(fr["hist"], [d * _L + iota], ones)
                return 0

            lax.fori_loop(0, _C, hist_body, 0)
            scan_hists()

            def reorder_body(c, _, src=src, srcI=srcI, dstK=dstK, dstI=dstI,
                             sh=sh):
                for fr in frames:
                    k = fr[src][pl.ds(c * _L, _L)]
                    pidx = fr[srcI][pl.ds(c * _L, _L)]
                    d = (k >> sh) & 15
                    hidx = d * _L + iota
                    q = plsc.load_gather(fr["hist"], [hidx])
                    plsc.addupdate_scatter(fr["hist"], [hidx], ones)
                    dst = (q & 63) * _L + (q >> 6)
                    plsc.store_scatter(fr[dstK], [dst], k)
                    plsc.store_scatter(fr[dstI], [dst], pidx)
                return 0

            lax.fori_loop(0, _C, reorder_body, 0)

        zero_hists()

        def hist7_body(c, _):
            for fr in frames:
                k = fr["key_b"][pl.ds(c * _L, _L)]
                d = (k >> 28) & 15
                plsc.addupdate_scatter(fr["hist"], [d * _L + iota], ones)
            return 0

        lax.fori_loop(0, _C, hist7_body, 0)
        scan_hists()

        def apply7_body(c, _):
            for fr in frames:
                k = fr["key_b"][pl.ds(c * _L, _L)]
                pidx = fr["idx_b"][pl.ds(c * _L, _L)]
                d = (k >> 28) & 15
                hidx = d * _L + iota
                q = plsc.load_gather(fr["hist"], [hidx])
                plsc.addupdate_scatter(fr["hist"], [hidx], ones)
                tv = plsc.load_gather(fr["t_v"], [q])
                plsc.store_scatter(fr["o_v"], [pidx], tv)
            return 0

        lax.fori_loop(0, _C, apply7_body, 0)
        for u, fr in enumerate(frames):
            pltpu.sync_copy(fr["o_v"], out_hbm.at[f0 + u])
        return 0

    lax.fori_loop(0, _FPW // 2, pair_body, 0)


@jax.jit
def _run(scores_flat, tbl):
    mesh = plsc.VectorSubcoreMesh(core_axis_name="c", subcore_axis_name="s")
    k = pl.kernel(
        _sc_body,
        mesh=mesh,
        compiler_params=pltpu.CompilerParams(
            needs_layout_passes=False,
            use_tc_tiling_on_sc=False,
        ),
        out_type=jax.ShapeDtypeStruct((_F, _N), jnp.float32),
        scratch_types=[
            pltpu.VMEM((_N,), jnp.float32),
            pltpu.VMEM((_N,), jnp.float32),
            pltpu.VMEM((_N,), jnp.float32),
            pltpu.VMEM((_N,), jnp.int32),
            pltpu.VMEM((_N,), jnp.int32),
            pltpu.VMEM((_N,), jnp.int32),
            pltpu.VMEM((_N,), jnp.int32),
            pltpu.VMEM((16 * _L,), jnp.int32),
        ] * 2,
    )
    return k(scores_flat, tbl)


def kernel(scores, mask_ratio):
    del mask_ratio
    scores_flat = scores.reshape(_F, _N)
    out = _run(scores_flat, jnp.asarray(_TBL))
    return (out > 0.5).reshape(_B, _T, 32, 32)


# speedup vs baseline: 1.0719x; 1.0680x over previous
import functools

import numpy as np
import jax
from jax import lax
import jax.numpy as jnp
from jax.experimental import pallas as pl
from jax.experimental.pallas import tpu as pltpu
from jax.experimental.pallas import tpu_sc as plsc

_B, _T, _N = 16, 16, 1024
_F = _B * _T
_NUM_STRATA = 4
_GRADIENT_STRENGTH = 0.15
_REGION_RATIOS = [0.4, 0.3, 0.2, 0.1]
_MASK_RATIO_STATIC = 0.75
_L = 16
_C = _N // _L
_NW = 32
_FPW = _F // _NW


def _rotl(x, d):
    return ((x << np.uint32(d)) | (x >> np.uint32(32 - d))).astype(np.uint32)


_TF_ROTS = ((13, 15, 26, 6), (17, 29, 16, 24))


def _threefry2x32(k0, k1, x0, x1):
    x0 = np.atleast_1d(x0).astype(np.uint32)
    x1 = np.atleast_1d(x1).astype(np.uint32)
    ks = (np.uint32(k0), np.uint32(k1),
          np.uint32(np.uint32(k0) ^ np.uint32(k1) ^ np.uint32(0x1BD11BDA)))
    x0 += ks[0]
    x1 += ks[1]
    inj = ((ks[1], ks[2]), (ks[2], ks[0]), (ks[0], ks[1]),
           (ks[1], ks[2]), (ks[2], ks[0]))
    for r in range(5):
        for rot in _TF_ROTS[r % 2]:
            x0 = (x0 + x1).astype(np.uint32)
            x1 = _rotl(x1, rot) ^ x0
        a, b = inj[r]
        x0 = (x0 + a).astype(np.uint32)
        x1 = (x1 + b + np.uint32(r + 1)).astype(np.uint32)
    return x0, x1


def _tf_fold_in(keypair, data):
    x0, x1 = _threefry2x32(keypair[0], keypair[1],
                           np.zeros(1, np.uint32), np.full(1, data, np.uint32))
    return (x0[0], x1[0])


def _tf_uniform(keypair, shape):
    size = int(np.prod(shape))
    i64 = np.arange(size, dtype=np.uint64)
    hi = (i64 >> np.uint64(32)).astype(np.uint32)
    lo = (i64 & np.uint64(0xFFFFFFFF)).astype(np.uint32)
    b0, b1 = _threefry2x32(keypair[0], keypair[1], hi, lo)
    bits = b0 ^ b1
    floats = (bits >> np.uint32(9)) | np.uint32(0x3F800000)
    return (floats.view(np.float32) - np.float32(1.0)).reshape(shape)


def _layer_ratio_list(mask_ratio):
    step = _GRADIENT_STRENGTH
    base = mask_ratio - (_NUM_STRATA - 1) * step / 2
    ratios = []
    for i in range(_NUM_STRATA):
        r = base + (_NUM_STRATA - 1 - i) * step
        r = max(0.0, min(0.9, r))
        ratios.append(r)
    weighted = sum(r * w for r, w in zip(ratios, _REGION_RATIOS))
    if weighted > 0:
        scale = mask_ratio / weighted
        ratios = [r * scale for r in ratios]
    return ratios


@functools.lru_cache(maxsize=1)
def _rank_table_f32():
    ratios = _layer_ratio_list(_MASK_RATIO_STATIC)
    sizes = [max(1, int(_N * r)) for r in _REGION_RATIOS]
    diff = _N - sum(sizes)
    if diff != 0:
        mi = sizes.index(max(sizes))
        sizes[mi] += diff
    key = (np.uint32(0), np.uint32(42))
    tbl = np.zeros((_B, _T, _N), np.float32)
    bI = np.arange(_B)[:, None, None]
    tI = np.arange(_T)[None, :, None]
    start = 0
    for j, layer_idx in enumerate(range(_NUM_STRATA - 1, -1, -1)):
        size = sizes[layer_idx]
        start_j = start
        start += size
        if size == 0:
            continue
        num = min(int(size * ratios[layer_idx]), size)
        if num <= 0:
            continue
        u = _tf_uniform(_tf_fold_in(key, j), (_B, _T, size))
        perm = np.argsort(u, axis=-1, kind="stable")[:, :, :num]
        tbl[bI, tI, start_j + perm] = 1.0
    return tbl.reshape(_F, _N)


_TBL = _rank_table_f32()


def _sc_body(scores_hbm, tbl_hbm, out_hbm,
             s_v, t_v, o_v,
             key_a0, key_b0, idx_a0, idx_b0, hist0,
             key_a1, key_b1, idx_a1, idx_b1, hist1):
    iota = lax.iota(jnp.int32, _L)
    ones = jnp.ones((_L,), jnp.int32)
    wid = lax.axis_index("s") * 2 + lax.axis_index("c")
    pltpu.sync_copy(scores_hbm.at[wid], s_v)
    pltpu.sync_copy(tbl_hbm.at[wid], t_v)
    fr0 = dict(key_a=key_a0, key_b=key_b0, idx_a=idx_a0, idx_b=idx_b0,
               hist=hist0)
    fr1 = dict(key_a=key_a1, key_b=key_b1, idx_a=idx_a1, idx_b=idx_b1,
               hist=hist1)
    frames = (fr0, fr1)

    def key0_at(fr, c):
        s = plsc.load_gather(s_v, [fr["base"] + iota * _C + c])
        b = lax.bitcast_convert_type(s, jnp.int32)
        return ~jnp.where(b >= 0, b ^ jnp.int32(-(2 ** 31)), ~b)

    def pair_body(i, _):
        fr0["base"] = i * 2 * _N
        fr1["base"] = (i * 2 + 1) * _N

        def zero_hists():
            for j in range(16):
                for fr in frames:
                    fr["hist"][pl.ds(j * _L, _L)] = jnp.zeros((_L,),
                                                              jnp.int32)

        def scan_hists():
            carries = [jnp.int32(0), jnp.int32(0)]
            for j in range(16):
                for u, fr in enumerate(frames):
                    h = fr["hist"][pl.ds(j * _L, _L)]
                    inc = jnp.cumsum(h)
                    fr["hist"][pl.ds(j * _L, _L)] = inc - h + carries[u]
                    carries[u] = carries[u] + jnp.sum(h)

        zero_hists()

        def hist0_body(c4, _):
            for c in range(0, 4):
                for fr in frames:
                    k = key0_at(fr, c4 * 4 + c)
                    plsc.addupdate_scatter(fr["hist"],
                                           [(k & 15) * _L + iota], ones)
            return 0

        lax.fori_loop(0, _C // 4, hist0_body, 0)
        scan_hists()

        def reorder0_body(c4, _):
            for c in range(0, 4):
                for fr in frames:
                    cc = c4 * 4 + c
                    k = key0_at(fr, cc)
                    hidx = (k & 15) * _L + iota
                    q = plsc.load_gather(fr["hist"], [hidx])
                    plsc.addupdate_scatter(fr["hist"], [hidx], ones)
                    dst = (q & 63) * _L + (q >> 6)
                    plsc.store_scatter(fr["key_b"], [dst], k)
                    plsc.store_scatter(fr["idx_b"], [dst], iota * _C + cc)
            return 0

        lax.fori_loop(0, _C // 4, reorder0_body, 0)

        for p in range(1, 7):
            sh = 4 * p
            src = "key_a" if p % 2 == 0 else "key_b"
            srcI = "idx_a" if p % 2 == 0 else "idx_b"
            dstK = "key_b" if p % 2 == 0 else "key_a"
            dstI = "idx_b" if p % 2 == 0 else "idx_a"
            zero_hists()

            def hist_body(c4, _, src=src, sh=sh):
                for c in range(0, 4):
                    for fr in frames:
                        k = fr[src][pl.ds((c4 * 4 + c) * _L, _L)]
                        d = (k >> sh) & 15
                        plsc.addupdate_scatter(fr["hist"], [d * _L + iota],
                                               ones)
                return 0

            lax.fori_loop(0, _C // 4, hist_body, 0)
            scan_hists()

            def reorder_body(c4, _, src=src, srcI=srcI, dstK=dstK, dstI=dstI,
                             sh=sh):
                for c in range(0, 4):
                    for fr in frames:
                        cc = c4 * 4 + c
                        k = fr[src][pl.ds(cc * _L, _L)]
                        pidx = fr[srcI][pl.ds(cc * _L, _L)]
                        d = (k >> sh) & 15
                        hidx = d * _L + iota
                        q = plsc.load_gather(fr["hist"], [hidx])
                        plsc.addupdate_scatter(fr["hist"], [hidx], ones)
                        dst = (q & 63) * _L + (q >> 6)
                        plsc.store_scatter(fr[dstK], [dst], k)
                        plsc.store_scatter(fr[dstI], [dst], pidx)
                return 0

            lax.fori_loop(0, _C // 4, reorder_body, 0)

        zero_hists()

        def hist7_body(c4, _):
            for c in range(0, 4):
                for fr in frames:
                    k = fr["key_b"][pl.ds((c4 * 4 + c) * _L, _L)]
                    d = (k >> 28) & 15
                    plsc.addupdate_scatter(fr["hist"], [d * _L + iota], ones)
            return 0

        lax.fori_loop(0, _C // 4, hist7_body, 0)
        scan_hists()

        def apply7_body(c4, _):
            for c in range(0, 4):
                for fr in frames:
                    cc = c4 * 4 + c
                    k = fr["key_b"][pl.ds(cc * _L, _L)]
                    pidx = fr["idx_b"][pl.ds(cc * _L, _L)]
                    d = (k >> 28) & 15
                    hidx = d * _L + iota
                    q = plsc.load_gather(fr["hist"], [hidx])
                    plsc.addupdate_scatter(fr["hist"], [hidx], ones)
                    tv = plsc.load_gather(t_v, [fr["base"] + q])
                    plsc.store_scatter(o_v, [fr["base"] + pidx], tv)
            return 0

        lax.fori_loop(0, _C // 4, apply7_body, 0)
        return 0

    lax.fori_loop(0, _FPW // 2, pair_body, 0)
    pltpu.sync_copy(o_v, out_hbm.at[wid])


@jax.jit
def _run(scores_flat, tbl):
    mesh = plsc.VectorSubcoreMesh(core_axis_name="c", subcore_axis_name="s")
    k = pl.kernel(
        _sc_body,
        mesh=mesh,
        compiler_params=pltpu.CompilerParams(
            needs_layout_passes=False,
            use_tc_tiling_on_sc=False,
        ),
        out_type=jax.ShapeDtypeStruct((_NW, _FPW * _N), jnp.float32),
        scratch_types=[
            pltpu.VMEM((_FPW * _N,), jnp.float32),
            pltpu.VMEM((_FPW * _N,), jnp.float32),
            pltpu.VMEM((_FPW * _N,), jnp.float32),
        ] + [
            pltpu.VMEM((_N,), jnp.int32),
            pltpu.VMEM((_N,), jnp.int32),
            pltpu.VMEM((_N,), jnp.int32),
            pltpu.VMEM((_N,), jnp.int32),
            pltpu.VMEM((16 * _L,), jnp.int32),
        ] * 2,
    )
    return k(scores_flat, tbl)


def kernel(scores, mask_ratio):
    del mask_ratio
    scores_flat = scores.reshape(_NW, _FPW * _N)
    out = _run(scores_flat, jnp.asarray(_TBL.reshape(_NW, _FPW * _N)))
    return (out > 0.5).reshape(_B, _T, 32, 32)


# speedup vs baseline: 1.3105x; 1.2226x over previous
import functools

import numpy as np
import jax
from jax import lax
import jax.numpy as jnp
from jax.experimental import pallas as pl
from jax.experimental.pallas import tpu as pltpu
from jax.experimental.pallas import tpu_sc as plsc

_B, _T, _N = 16, 16, 1024
_F = _B * _T
_NUM_STRATA = 4
_GRADIENT_STRENGTH = 0.15
_REGION_RATIOS = [0.4, 0.3, 0.2, 0.1]
_MASK_RATIO_STATIC = 0.75
_L = 16
_C = _N // _L
_NW = 32
_FPW = 5
_F_SC = _NW * _FPW
_F_TC = _F - _F_SC


def _rotl(x, d):
    return ((x << np.uint32(d)) | (x >> np.uint32(32 - d))).astype(np.uint32)


_TF_ROTS = ((13, 15, 26, 6), (17, 29, 16, 24))


def _threefry2x32(k0, k1, x0, x1):
    x0 = np.atleast_1d(x0).astype(np.uint32)
    x1 = np.atleast_1d(x1).astype(np.uint32)
    ks = (np.uint32(k0), np.uint32(k1),
          np.uint32(np.uint32(k0) ^ np.uint32(k1) ^ np.uint32(0x1BD11BDA)))
    x0 += ks[0]
    x1 += ks[1]
    inj = ((ks[1], ks[2]), (ks[2], ks[0]), (ks[0], ks[1]),
           (ks[1], ks[2]), (ks[2], ks[0]))
    for r in range(5):
        for rot in _TF_ROTS[r % 2]:
            x0 = (x0 + x1).astype(np.uint32)
            x1 = _rotl(x1, rot) ^ x0
        a, b = inj[r]
        x0 = (x0 + a).astype(np.uint32)
        x1 = (x1 + b + np.uint32(r + 1)).astype(np.uint32)
    return x0, x1


def _tf_fold_in(keypair, data):
    x0, x1 = _threefry2x32(keypair[0], keypair[1],
                           np.zeros(1, np.uint32), np.full(1, data, np.uint32))
    return (x0[0], x1[0])


def _tf_uniform(keypair, shape):
    size = int(np.prod(shape))
    i64 = np.arange(size, dtype=np.uint64)
    hi = (i64 >> np.uint64(32)).astype(np.uint32)
    lo = (i64 & np.uint64(0xFFFFFFFF)).astype(np.uint32)
    b0, b1 = _threefry2x32(keypair[0], keypair[1], hi, lo)
    bits = b0 ^ b1
    floats = (bits >> np.uint32(9)) | np.uint32(0x3F800000)
    return (floats.view(np.float32) - np.float32(1.0)).reshape(shape)


def _layer_ratio_list(mask_ratio):
    step = _GRADIENT_STRENGTH
    base = mask_ratio - (_NUM_STRATA - 1) * step / 2
    ratios = []
    for i in range(_NUM_STRATA):
        r = base + (_NUM_STRATA - 1 - i) * step
        r = max(0.0, min(0.9, r))
        ratios.append(r)
    weighted = sum(r * w for r, w in zip(ratios, _REGION_RATIOS))
    if weighted > 0:
        scale = mask_ratio / weighted
        ratios = [r * scale for r in ratios]
    return ratios


@functools.lru_cache(maxsize=1)
def _rank_table_f32():
    ratios = _layer_ratio_list(_MASK_RATIO_STATIC)
    sizes = [max(1, int(_N * r)) for r in _REGION_RATIOS]
    diff = _N - sum(sizes)
    if diff != 0:
        mi = sizes.index(max(sizes))
        sizes[mi] += diff
    key = (np.uint32(0), np.uint32(42))
    tbl = np.zeros((_B, _T, _N), np.float32)
    bI = np.arange(_B)[:, None, None]
    tI = np.arange(_T)[None, :, None]
    start = 0
    for j, layer_idx in enumerate(range(_NUM_STRATA - 1, -1, -1)):
        size = sizes[layer_idx]
        start_j = start
        start += size
        if size == 0:
            continue
        num = min(int(size * ratios[layer_idx]), size)
        if num <= 0:
            continue
        u = _tf_uniform(_tf_fold_in(key, j), (_B, _T, size))
        perm = np.argsort(u, axis=-1, kind="stable")[:, :, :num]
        tbl[bI, tI, start_j + perm] = 1.0
    return tbl.reshape(_F, _N)


_TBL = _rank_table_f32()


@functools.lru_cache(maxsize=1)
def _packed_words_tc():
    flat = _TBL[_F_SC:].astype(bool).reshape(_F_TC, _N // 32, 32)
    words = np.zeros((_F_TC, _N // 32), np.uint32)
    for b in range(32):
        words |= flat[:, :, b].astype(np.uint32) << np.uint32(b)
    return words.view(np.int32).reshape(_F_TC, _N // 32, 1)


def _tc_rank_mask_kernel(s_ref, w_ref, o_ref):
    srow = s_ref[0]
    scol = jnp.transpose(srow)
    im = lax.broadcasted_iota(jnp.int32, (_N, _N), 0)
    i_n = lax.broadcasted_iota(jnp.int32, (_N, _N), 1)
    gt = (scol > srow) | ((scol == srow) & (im < i_n))
    rank = jnp.sum(gt.astype(jnp.float32), axis=0, keepdims=True)
    rank = rank.astype(jnp.int32)
    w_idx = rank >> 5
    b_idx = rank & 31
    words = w_ref[0]
    iw = lax.broadcasted_iota(jnp.int32, (32, _N), 0)
    sel = iw == w_idx
    wsel = jnp.sum(jnp.where(sel, words, 0), axis=0, keepdims=True)
    bit = lax.shift_right_logical(wsel, b_idx) & 1
    o_ref[0] = bit.astype(jnp.float32)


def _sc_body(scores_hbm, tbl_hbm, out_hbm,
             s_v, t_v, o_v, key_a, key_b, idx_a, idx_b, hist):
    iota = lax.iota(jnp.int32, _L)
    ones = jnp.ones((_L,), jnp.int32)
    wid = lax.axis_index("s") * 2 + lax.axis_index("c")
    pltpu.sync_copy(scores_hbm.at[wid], s_v)
    pltpu.sync_copy(tbl_hbm.at[wid], t_v)

    def frame_body(i, _):
        base = i * _N

        def key0_at(c):
            s = plsc.load_gather(s_v, [base + iota * _C + c])
            b = lax.bitcast_convert_type(s, jnp.int32)
            return ~jnp.where(b >= 0, b ^ jnp.int32(-(2 ** 31)), ~b)

        def zero_hist():
            for j in range(16):
                hist[pl.ds(j * _L, _L)] = jnp.zeros((_L,), jnp.int32)

        def scan_hist():
            carry = jnp.int32(0)
            for j in range(16):
                h = hist[pl.ds(j * _L, _L)]
                inc = jnp.cumsum(h)
                hist[pl.ds(j * _L, _L)] = inc - h + carry
                carry = carry + jnp.sum(h)

        zero_hist()

        def hist0_body(c4, _):
            for c in range(4):
                k = key0_at(c4 * 4 + c)
                plsc.addupdate_scatter(hist, [(k & 15) * _L + iota], ones)
            return 0

        lax.fori_loop(0, _C // 4, hist0_body, 0)
        scan_hist()

        def reorder0_body(c4, _):
            for c in range(4):
                cc = c4 * 4 + c
                k = key0_at(cc)
                hidx = (k & 15) * _L + iota
                q = plsc.load_gather(hist, [hidx])
                plsc.addupdate_scatter(hist, [hidx], ones)
                dst = (q & 63) * _L + (q >> 6)
                plsc.store_scatter(key_b, [dst], k)
                plsc.store_scatter(idx_b, [dst], iota * _C + cc)
            return 0

        lax.fori_loop(0, _C // 4, reorder0_body, 0)

        bufs = [(key_a, idx_a), (key_b, idx_b)]
        for p in range(1, 7):
            sh = 4 * p
            in_key, in_idx = bufs[p % 2]
            out_key, out_idx = bufs[(p + 1) % 2]
            zero_hist()

            def hist_body(c4, _, in_key=in_key, sh=sh):
                for c in range(4):
                    k = in_key[pl.ds((c4 * 4 + c) * _L, _L)]
                    d = (k >> sh) & 15
                    plsc.addupdate_scatter(hist, [d * _L + iota], ones)
                return 0

            lax.fori_loop(0, _C // 4, hist_body, 0)
            scan_hist()

            def reorder_body(c4, _, in_key=in_key, in_idx=in_idx,
                             out_key=out_key, out_idx=out_idx, sh=sh):
                for c in range(4):
                    cc = c4 * 4 + c
                    k = in_key[pl.ds(cc * _L, _L)]
                    pidx = in_idx[pl.ds(cc * _L, _L)]
                    d = (k >> sh) & 15
                    hidx = d * _L + iota
                    q = plsc.load_gather(hist, [hidx])
                    plsc.addupdate_scatter(hist, [hidx], ones)
                    dst = (q & 63) * _L + (q >> 6)
                    plsc.store_scatter(out_key, [dst], k)
                    plsc.store_scatter(out_idx, [dst], pidx)
                return 0

            lax.fori_loop(0, _C // 4, reorder_body, 0)

        zero_hist()

        def hist7_body(c4, _):
            for c in range(4):
                k = key_b[pl.ds((c4 * 4 + c) * _L, _L)]
                d = (k >> 28) & 15
                plsc.addupdate_scatter(hist, [d * _L + iota], ones)
            return 0

        lax.fori_loop(0, _C // 4, hist7_body, 0)
        scan_hist()

        def apply7_body(c4, _):
            for c in range(4):
                cc = c4 * 4 + c
                k = key_b[pl.ds(cc * _L, _L)]
                pidx = idx_b[pl.ds(cc * _L, _L)]
                d = (k >> 28) & 15
                hidx = d * _L + iota
                q = plsc.load_gather(hist, [hidx])
                plsc.addupdate_scatter(hist, [hidx], ones)
                tv = plsc.load_gather(t_v, [base + q])
                plsc.store_scatter(o_v, [base + pidx], tv)
            return 0

        lax.fori_loop(0, _C // 4, apply7_body, 0)
        return 0

    lax.fori_loop(0, _FPW, frame_body, 0)
    pltpu.sync_copy(o_v, out_hbm.at[wid])


@jax.jit
def _run(scores_sc, tbl_sc, scores_tc, words_tc):
    mesh = plsc.VectorSubcoreMesh(core_axis_name="c", subcore_axis_name="s")
    sc_k = pl.kernel(
        _sc_body,
        mesh=mesh,
        compiler_params=pltpu.CompilerParams(
            needs_layout_passes=False,
            use_tc_tiling_on_sc=False,
        ),
        out_type=jax.ShapeDtypeStruct((_NW, _FPW * _N), jnp.float32),
        scratch_types=[
            pltpu.VMEM((_FPW * _N,), jnp.float32),
            pltpu.VMEM((_FPW * _N,), jnp.float32),
            pltpu.VMEM((_FPW * _N,), jnp.float32),
            pltpu.VMEM((_N,), jnp.int32),
            pltpu.VMEM((_N,), jnp.int32),
            pltpu.VMEM((_N,), jnp.int32),
            pltpu.VMEM((_N,), jnp.int32),
            pltpu.VMEM((16 * _L,), jnp.int32),
        ],
    )
    out_sc = sc_k(scores_sc, tbl_sc)
    out_tc = pl.pallas_call(
        _tc_rank_mask_kernel,
        grid=(_F_TC,),
        in_specs=[
            pl.BlockSpec((1, 1, _N), lambda i: (i, 0, 0)),
            pl.BlockSpec((1, _N // 32, 1), lambda i: (i, 0, 0)),
        ],
        out_specs=pl.BlockSpec((1, 1, _N), lambda i: (i, 0, 0)),
        out_shape=jax.ShapeDtypeStruct((_F_TC, 1, _N), jnp.float32),
        compiler_params=pltpu.CompilerParams(
            dimension_semantics=("parallel",),
        ),
    )(scores_tc, words_tc)
    full = jnp.concatenate(
        [out_sc.reshape(_F_SC, _N), out_tc.reshape(_F_TC, _N)], axis=0)
    return full > 0.5


def kernel(scores, mask_ratio):
    del mask_ratio
    scores_flat = scores.reshape(_F, _N)
    out = _run(
        scores_flat[:_F_SC].reshape(_NW, _FPW * _N),
        jnp.asarray(_TBL[:_F_SC].reshape(_NW, _FPW * _N)),
        scores_flat[_F_SC:].reshape(_F_TC, 1, _N),
        jnp.asarray(_packed_words_tc()),
    )
    return out.reshape(_B, _T, 32, 32)
